# baseline (speedup 1.0000x reference)
import functools

import jax
import jax.numpy as jnp
from jax import lax
from jax.experimental import pallas as pl
from jax.experimental.pallas import tpu as pltpu
from jax.experimental.pallas import tpu_sc as plsc

LANES = 16
NC = 2
NS = 16
NW = NC * NS


def _iota16():
    return lax.iota(jnp.int32, LANES)


@functools.lru_cache(maxsize=None)
def _make_detile(vocab, embed):
    KC = 4
    CW = 128 * KC
    TCOLS = vocab // 128
    NCHUNK = TCOLS // KC
    TAIL = vocab - NCHUNK * CW
    NBUF = 3
    DG = 1
    assert embed == 32 and TAIL % 4 == 0
    mesh = plsc.VectorSubcoreMesh(core_axis_name="c", subcore_axis_name="s")

    @functools.partial(
        pl.kernel,
        mesh=mesh,
        compiler_params=pltpu.CompilerParams(needs_layout_passes=False),
        out_type=jax.ShapeDtypeStruct((vocab * embed // 128, 128), jnp.float32),
        scratch_types=(
            [pltpu.VMEM((32, CW), jnp.float32) for _ in range(NBUF)]
            + [pltpu.VMEM((CW // 4, 128), jnp.float32) for _ in range(NBUF)]
            + ([pltpu.VMEM((32, TAIL), jnp.float32),
                pltpu.VMEM((TAIL * 32 // 128, 128), jnp.float32)] if TAIL else [])
            + [pltpu.SemaphoreType.DMA for _ in range(2 * NBUF)]
        ),
    )
    def detile(tokT_hbm, out_hbm, *rest):
        vin = rest[:NBUF]
        vout = rest[NBUF:2 * NBUF]
        ntail = 2 if TAIL else 0
        if TAIL:
            tin, tout = rest[2 * NBUF:2 * NBUF + 2]
        gsems = rest[2 * NBUF + ntail:3 * NBUF + ntail]
        ssems = rest[3 * NBUF + ntail:4 * NBUF + ntail]

        wid = lax.axis_index("s") * NC + lax.axis_index("c")
        iota = _iota16()
        e_vecs = ((0, iota), (1, iota + LANES))

        def chunk_of(k):
            return k * NW + wid

        def start_read(k, b):
            pltpu.make_async_copy(
                tokT_hbm.at[:, pl.ds(chunk_of(k) * CW, CW)], vin[b], gsems[b]
            ).start()

        def wait_read(b):
            pltpu.make_async_copy(
                tokT_hbm.at[:, pl.ds(0, CW)], vin[b], gsems[b]
            ).wait()

        def start_write(k, b):
            pltpu.make_async_copy(
                vout[b], out_hbm.at[pl.ds(chunk_of(k) * (CW // 4), CW // 4)],
                ssems[b],
            ).start()

        def wait_write(b):
            pltpu.make_async_copy(
                vout[b], out_hbm.at[pl.ds(0, CW // 4)], ssems[b]
            ).wait()

        def transpose(b):
            @plsc.parallel_loop(0, CW // 4, unroll=4)
            def body(r):
                for q in range(4):
                    t_vec = jnp.full((LANES,), r * 4 + q, jnp.int32)
                    for h, e_vec in e_vecs:
                        vals = plsc.load_gather(vin[b], [e_vec, t_vec])
                        vout[b][r, pl.ds(q * 32 + h * LANES, LANES)] = vals

        valid = (NCHUNK - 1 - wid) // NW + 1

        for b in range(DG):
            @pl.when(b < valid)
            def _(b=b):
                start_read(b, b)

        def body(k, carry):
            for bb in range(NBUF):
                @pl.when(lax.rem(k, NBUF) == bb)
                def _(bb=bb):
                    nxt = k + DG
                    bn = (bb + DG) % NBUF

                    @pl.when(nxt < valid)
                    def _():
                        @pl.when(nxt >= NBUF)
                        def _():
                            wait_write(bn)
                        start_read(nxt, bn)

                    wait_read(bb)
                    transpose(bb)
                    start_write(k, bb)
            return carry

        lax.fori_loop(0, valid, body, 0)

        for j in range(NBUF):
            @pl.when(valid > j)
            def _(j=j):
                for bb in range(NBUF):
                    @pl.when(lax.rem(valid - 1 - j, NBUF) == bb)
                    def _(bb=bb):
                        wait_write(bb)

        if TAIL:
            @pl.when(wid == 0)
            def _():
                pltpu.async_copy(
                    tokT_hbm.at[:, pl.ds(NCHUNK * CW, TAIL)], tin, gsems[0]
                ).wait()

                @plsc.parallel_loop(0, TAIL * 32 // 128, unroll=4)
                def tail_body(r):
                    for q in range(4):
                        t_vec = jnp.full((LANES,), r * 4 + q, jnp.int32)
                        for h, e_vec in e_vecs:
                            vals = plsc.load_gather(tin, [e_vec, t_vec])
                            tout[r, pl.ds(q * 32 + h * LANES, LANES)] = vals

                pltpu.async_copy(
                    tout,
                    out_hbm.at[pl.ds(NCHUNK * CW // 4, TAIL * 32 // 128)],
                    gsems[0],
                ).wait()

    return detile


@functools.lru_cache(maxsize=None)
def _make_emb(batch, maxlen, embed, vocab):
    RPW = batch // NW
    TB = 4
    NBUF = 2
    DG = 1
    NSTEP = maxlen // TB
    assert RPW == 128 and embed == 32 and maxlen % TB == 0

    mesh = plsc.VectorSubcoreMesh(core_axis_name="c", subcore_axis_name="s")

    @functools.partial(
        pl.kernel,
        mesh=mesh,
        compiler_params=pltpu.CompilerParams(use_tc_tiling_on_sc=False,
                                             needs_layout_passes=False),
        out_type=jax.ShapeDtypeStruct((maxlen, embed // 8, batch // 128, 8, 128),
                                      jnp.float32),
        scratch_types=(
            [pltpu.VMEM((maxlen, RPW), jnp.int32),
             pltpu.VMEM((NSTEP, TB * RPW), jnp.int32),
             pltpu.VMEM((maxlen, embed), jnp.float32)]
            + [pltpu.VMEM((TB * RPW, embed), jnp.float32) for _ in range(NBUF)]
            + [pltpu.VMEM((TB, embed // 8, 1, 8, 128), jnp.float32)
               for _ in range(NBUF)]
            + [pltpu.SemaphoreType.DMA for _ in range(2 * NBUF + 1)]
        ),
    )
    def emb(xT_hbm, tok_hbm, pos_hbm, out_hbm, idx2d_v, idxT_v, pat_v, *rest):
        gbuf = rest[:NBUF]
        obuf = rest[NBUF:2 * NBUF]
        gsems = rest[2 * NBUF:3 * NBUF]
        ssems = rest[3 * NBUF:4 * NBUF]
        lsem = rest[4 * NBUF]

        wid = lax.axis_index("s") * NC + lax.axis_index("c")
        base = wid * RPW
        iota = _iota16()
        b_vecs = [j * LANES + iota for j in range(RPW // LANES)]

        pltpu.async_copy(xT_hbm.at[:, pl.ds(base, RPW)], idx2d_v, lsem).wait()
        pltpu.async_copy(pos_hbm, pat_v, lsem).wait()

        @plsc.parallel_loop(0, maxlen, unroll=4)
        def repack(t):
            o = (t & (TB - 1)) * RPW
            for j in range(RPW // LANES):
                idxT_v[t >> 2, pl.ds(o + j * LANES, LANES)] = (
                    idx2d_v[t, pl.ds(j * LANES, LANES)])

        def start_gather(s, b):
            pltpu.make_async_copy(
                tok_hbm.at[idxT_v.at[s]], gbuf[b], gsems[b]
            ).start()

        def wait_gather(b):
            pltpu.make_async_copy(
                tok_hbm.at[idxT_v.at[0]], gbuf[b], gsems[b]
            ).wait()

        def start_scatter(s, b):
            pltpu.make_async_copy(
                obuf[b], out_hbm.at[pl.ds(s * TB, TB), :, pl.ds(wid, 1)],
                ssems[b],
            ).start()

        def wait_scatter(b):
            pltpu.make_async_copy(
                obuf[b], out_hbm.at[pl.ds(0, TB), :, pl.ds(wid, 1)], ssems[b]
            ).wait()

        def transpose_add(s, b):
            @plsc.parallel_loop(0, TB * embed, unroll=4)
            def ebody(j):
                tt = j >> 5
                e = j & (embed - 1)
                e_vec = jnp.full((LANES,), e, jnp.int32)
                pv = plsc.load_gather(
                    pat_v, [jnp.full((LANES,), s * TB + tt, jnp.int32), e_vec])
                for g in range(RPW // LANES):
                    vals = plsc.load_gather(
                        gbuf[b], [tt * RPW + b_vecs[g], e_vec])
                    obuf[b][tt, e >> 3, 0, e & 7, pl.ds(g * LANES, LANES)] = (
                        vals + pv)

        for b in range(DG):
            start_gather(b, b)

        def outer(i, carry):
            s0 = i * NBUF
            for b in range(NBUF):
                s = s0 + b
                nxt = s + DG
                bn = (b + DG) % NBUF

                @pl.when(nxt < NSTEP)
                def _(nxt=nxt, bn=bn):
                    @pl.when(nxt >= NBUF)
                    def _():
                        wait_scatter(bn)
                    start_gather(nxt, bn)

                wait_gather(b)
                transpose_add(s, b)
                start_scatter(s, b)
            return carry

        lax.fori_loop(0, NSTEP // NBUF, outer, 0)

        for b in range(NBUF):
            wait_scatter(b)

    return emb


def kernel(x, token_table, pos_table):
    batch, maxlen = x.shape
    vocab, embed = token_table.shape
    tok_dense = _make_detile(vocab, embed)(token_table.T)
    tok_lin = tok_dense.reshape(vocab, embed)
    out5 = _make_emb(batch, maxlen, embed, vocab)(
        x.astype(jnp.int32).T, tok_lin, pos_table
    )
    return out5.transpose(2, 4, 0, 1, 3).reshape(batch, maxlen, embed)


# speedup vs baseline: 1.0056x; 1.0056x over previous
import functools

import jax
import jax.numpy as jnp
from jax import lax
from jax.experimental import pallas as pl
from jax.experimental.pallas import tpu as pltpu
from jax.experimental.pallas import tpu_sc as plsc

LANES = 16
NC = 2
NS = 16
NW = NC * NS


def _iota16():
    return lax.iota(jnp.int32, LANES)


@functools.lru_cache(maxsize=None)
def _make_detile(vocab, embed):
    KC = 4
    CW = 128 * KC
    TCOLS = vocab // 128
    NCHUNK = TCOLS // KC
    TAIL = vocab - NCHUNK * CW
    NBUF = 3
    DG = 1
    assert embed == 32 and TAIL % 4 == 0
    mesh = plsc.VectorSubcoreMesh(core_axis_name="c", subcore_axis_name="s")

    @functools.partial(
        pl.kernel,
        mesh=mesh,
        compiler_params=pltpu.CompilerParams(needs_layout_passes=False),
        out_type=jax.ShapeDtypeStruct((vocab * embed // 128, 128), jnp.float32),
        scratch_types=(
            [pltpu.VMEM((32, CW), jnp.float32) for _ in range(NBUF)]
            + [pltpu.VMEM((CW // 4, 128), jnp.float32) for _ in range(NBUF)]
            + ([pltpu.VMEM((32, TAIL), jnp.float32),
                pltpu.VMEM((TAIL * 32 // 128, 128), jnp.float32)] if TAIL else [])
            + [pltpu.SemaphoreType.DMA for _ in range(2 * NBUF)]
        ),
    )
    def detile(tokT_hbm, out_hbm, *rest):
        vin = rest[:NBUF]
        vout = rest[NBUF:2 * NBUF]
        ntail = 2 if TAIL else 0
        if TAIL:
            tin, tout = rest[2 * NBUF:2 * NBUF + 2]
        gsems = rest[2 * NBUF + ntail:3 * NBUF + ntail]
        ssems = rest[3 * NBUF + ntail:4 * NBUF + ntail]

        wid = lax.axis_index("s") * NC + lax.axis_index("c")
        iota = _iota16()
        e_vecs = ((0, iota), (1, iota + LANES))

        def chunk_of(k):
            return k * NW + wid

        def start_read(k, b):
            pltpu.make_async_copy(
                tokT_hbm.at[:, pl.ds(chunk_of(k) * CW, CW)], vin[b], gsems[b]
            ).start()

        def wait_read(b):
            pltpu.make_async_copy(
                tokT_hbm.at[:, pl.ds(0, CW)], vin[b], gsems[b]
            ).wait()

        def start_write(k, b):
            pltpu.make_async_copy(
                vout[b], out_hbm.at[pl.ds(chunk_of(k) * (CW // 4), CW // 4)],
                ssems[b],
            ).start()

        def wait_write(b):
            pltpu.make_async_copy(
                vout[b], out_hbm.at[pl.ds(0, CW // 4)], ssems[b]
            ).wait()

        def transpose(b):
            @plsc.parallel_loop(0, CW // 4, unroll=8)
            def body(r):
                for q in range(4):
                    t_vec = jnp.full((LANES,), r * 4 + q, jnp.int32)
                    for h, e_vec in e_vecs:
                        vals = plsc.load_gather(vin[b], [e_vec, t_vec])
                        vout[b][r, pl.ds(q * 32 + h * LANES, LANES)] = vals

        valid = (NCHUNK - 1 - wid) // NW + 1

        for b in range(DG):
            @pl.when(b < valid)
            def _(b=b):
                start_read(b, b)

        def body(k, carry):
            for bb in range(NBUF):
                @pl.when(lax.rem(k, NBUF) == bb)
                def _(bb=bb):
                    nxt = k + DG
                    bn = (bb + DG) % NBUF

                    @pl.when(nxt < valid)
                    def _():
                        @pl.when(nxt >= NBUF)
                        def _():
                            wait_write(bn)
                        start_read(nxt, bn)

                    wait_read(bb)
                    transpose(bb)
                    start_write(k, bb)
            return carry

        lax.fori_loop(0, valid, body, 0)

        for j in range(NBUF):
            @pl.when(valid > j)
            def _(j=j):
                for bb in range(NBUF):
                    @pl.when(lax.rem(valid - 1 - j, NBUF) == bb)
                    def _(bb=bb):
                        wait_write(bb)

        if TAIL:
            @pl.when(wid == 0)
            def _():
                pltpu.async_copy(
                    tokT_hbm.at[:, pl.ds(NCHUNK * CW, TAIL)], tin, gsems[0]
                ).wait()

                @plsc.parallel_loop(0, TAIL * 32 // 128, unroll=4)
                def tail_body(r):
                    for q in range(4):
                        t_vec = jnp.full((LANES,), r * 4 + q, jnp.int32)
                        for h, e_vec in e_vecs:
                            vals = plsc.load_gather(tin, [e_vec, t_vec])
                            tout[r, pl.ds(q * 32 + h * LANES, LANES)] = vals

                pltpu.async_copy(
                    tout,
                    out_hbm.at[pl.ds(NCHUNK * CW // 4, TAIL * 32 // 128)],
                    gsems[0],
                ).wait()

    return detile


@functools.lru_cache(maxsize=None)
def _make_emb(batch, maxlen, embed, vocab):
    RPW = batch // NW
    TB = 4
    NBUF = 2
    DG = 1
    NSTEP = maxlen // TB
    assert RPW == 128 and embed == 32 and maxlen % TB == 0

    mesh = plsc.VectorSubcoreMesh(core_axis_name="c", subcore_axis_name="s")

    @functools.partial(
        pl.kernel,
        mesh=mesh,
        compiler_params=pltpu.CompilerParams(use_tc_tiling_on_sc=False,
                                             needs_layout_passes=False),
        out_type=jax.ShapeDtypeStruct((maxlen, embed // 8, batch // 128, 8, 128),
                                      jnp.float32),
        scratch_types=(
            [pltpu.VMEM((maxlen, RPW), jnp.int32),
             pltpu.VMEM((NSTEP, TB * RPW), jnp.int32),
             pltpu.VMEM((maxlen, embed), jnp.float32)]
            + [pltpu.VMEM((TB * RPW, embed), jnp.float32) for _ in range(NBUF)]
            + [pltpu.VMEM((TB, embed // 8, 1, 8, 128), jnp.float32)
               for _ in range(NBUF)]
            + [pltpu.SemaphoreType.DMA for _ in range(2 * NBUF + 1)]
        ),
    )
    def emb(xT_hbm, tok_hbm, pos_hbm, out_hbm, idx2d_v, idxT_v, pat_v, *rest):
        gbuf = rest[:NBUF]
        obuf = rest[NBUF:2 * NBUF]
        gsems = rest[2 * NBUF:3 * NBUF]
        ssems = rest[3 * NBUF:4 * NBUF]
        lsem = rest[4 * NBUF]

        wid = lax.axis_index("s") * NC + lax.axis_index("c")
        base = wid * RPW
        iota = _iota16()
        b_vecs = [j * LANES + iota for j in range(RPW // LANES)]

        pltpu.async_copy(xT_hbm.at[:, pl.ds(base, RPW)], idx2d_v, lsem).wait()
        pltpu.async_copy(pos_hbm, pat_v, lsem).wait()

        @plsc.parallel_loop(0, maxlen, unroll=8)
        def repack(t):
            o = (t & (TB - 1)) * RPW
            for j in range(RPW // LANES):
                idxT_v[t >> 2, pl.ds(o + j * LANES, LANES)] = (
                    idx2d_v[t, pl.ds(j * LANES, LANES)])

        def start_gather(s, b):
            pltpu.make_async_copy(
                tok_hbm.at[idxT_v.at[s]], gbuf[b], gsems[b]
            ).start()

        def wait_gather(b):
            pltpu.make_async_copy(
                tok_hbm.at[idxT_v.at[0]], gbuf[b], gsems[b]
            ).wait()

        def start_scatter(s, b):
            pltpu.make_async_copy(
                obuf[b], out_hbm.at[pl.ds(s * TB, TB), :, pl.ds(wid, 1)],
                ssems[b],
            ).start()

        def wait_scatter(b):
            pltpu.make_async_copy(
                obuf[b], out_hbm.at[pl.ds(0, TB), :, pl.ds(wid, 1)], ssems[b]
            ).wait()

        def transpose_add(s, b):
            @plsc.parallel_loop(0, TB * embed, unroll=8)
            def ebody(j):
                tt = j >> 5
                e = j & (embed - 1)
                e_vec = jnp.full((LANES,), e, jnp.int32)
                pv = plsc.load_gather(
                    pat_v, [jnp.full((LANES,), s * TB + tt, jnp.int32), e_vec])
                for g in range(RPW // LANES):
                    vals = plsc.load_gather(
                        gbuf[b], [tt * RPW + b_vecs[g], e_vec])
                    obuf[b][tt, e >> 3, 0, e & 7, pl.ds(g * LANES, LANES)] = (
                        vals + pv)

        for b in range(DG):
            start_gather(b, b)

        def outer(i, carry):
            s0 = i * NBUF
            for b in range(NBUF):
                s = s0 + b
                nxt = s + DG
                bn = (b + DG) % NBUF

                @pl.when(nxt < NSTEP)
                def _(nxt=nxt, bn=bn):
                    @pl.when(nxt >= NBUF)
                    def _():
                        wait_scatter(bn)
                    start_gather(nxt, bn)

                wait_gather(b)
                transpose_add(s, b)
                start_scatter(s, b)
            return carry

        lax.fori_loop(0, NSTEP // NBUF, outer, 0)

        for b in range(NBUF):
            wait_scatter(b)

    return emb


def kernel(x, token_table, pos_table):
    batch, maxlen = x.shape
    vocab, embed = token_table.shape
    tok_dense = _make_detile(vocab, embed)(token_table.T)
    tok_lin = tok_dense.reshape(vocab, embed)
    out5 = _make_emb(batch, maxlen, embed, vocab)(
        x.astype(jnp.int32).T, tok_lin, pos_table
    )
    return out5.transpose(2, 4, 0, 1, 3).reshape(batch, maxlen, embed)


# speedup vs baseline: 1.8748x; 1.8643x over previous
import functools

import jax
import jax.numpy as jnp
from jax import lax
from jax.experimental import pallas as pl
from jax.experimental.pallas import tpu as pltpu
from jax.experimental.pallas import tpu_sc as plsc

LANES = 16
NC = 2
NS = 16
NW = NC * NS


def _iota16():
    return lax.iota(jnp.int32, LANES)


@functools.lru_cache(maxsize=None)
def _make_detile(vocab, embed):
    KC = 4
    CW = 128 * KC
    TCOLS = vocab // 128
    NCHUNK = TCOLS // KC
    TAIL = vocab - NCHUNK * CW
    NBUF = 3
    DG = 1
    assert embed == 32 and TAIL % 4 == 0
    mesh = plsc.VectorSubcoreMesh(core_axis_name="c", subcore_axis_name="s")

    @functools.partial(
        pl.kernel,
        mesh=mesh,
        compiler_params=pltpu.CompilerParams(needs_layout_passes=False),
        out_type=jax.ShapeDtypeStruct((vocab * embed // 128, 128), jnp.float32),
        scratch_types=(
            [pltpu.VMEM((32, CW), jnp.float32) for _ in range(NBUF)]
            + [pltpu.VMEM((CW // 4, 128), jnp.float32) for _ in range(NBUF)]
            + ([pltpu.VMEM((32, TAIL), jnp.float32),
                pltpu.VMEM((TAIL * 32 // 128, 128), jnp.float32)] if TAIL else [])
            + [pltpu.SemaphoreType.DMA for _ in range(2 * NBUF)]
        ),
    )
    def detile(tokT_hbm, out_hbm, *rest):
        vin = rest[:NBUF]
        vout = rest[NBUF:2 * NBUF]
        ntail = 2 if TAIL else 0
        if TAIL:
            tin, tout = rest[2 * NBUF:2 * NBUF + 2]
        gsems = rest[2 * NBUF + ntail:3 * NBUF + ntail]
        ssems = rest[3 * NBUF + ntail:4 * NBUF + ntail]

        wid = lax.axis_index("s") * NC + lax.axis_index("c")
        iota = _iota16()
        e_vecs = ((0, iota), (1, iota + LANES))

        def chunk_of(k):
            return k * NW + wid

        def start_read(k, b):
            pltpu.make_async_copy(
                tokT_hbm.at[:, pl.ds(chunk_of(k) * CW, CW)], vin[b], gsems[b]
            ).start()

        def wait_read(b):
            pltpu.make_async_copy(
                tokT_hbm.at[:, pl.ds(0, CW)], vin[b], gsems[b]
            ).wait()

        def start_write(k, b):
            pltpu.make_async_copy(
                vout[b], out_hbm.at[pl.ds(chunk_of(k) * (CW // 4), CW // 4)],
                ssems[b],
            ).start()

        def wait_write(b):
            pltpu.make_async_copy(
                vout[b], out_hbm.at[pl.ds(0, CW // 4)], ssems[b]
            ).wait()

        def transpose(b):
            @plsc.parallel_loop(0, CW // LANES, unroll=2)
            def body(m):
                t_vec = m * LANES + iota
                t32 = t_vec * 32
                for rot in range(32):
                    e_vec = (iota + rot) & 31
                    vals = plsc.load_gather(vin[b], [e_vec, t_vec])
                    flat = t32 + e_vec
                    plsc.store_scatter(
                        vout[b], [flat >> 7, flat & 127], vals)

        valid = (NCHUNK - 1 - wid) // NW + 1

        for b in range(DG):
            @pl.when(b < valid)
            def _(b=b):
                start_read(b, b)

        def body(k, carry):
            for bb in range(NBUF):
                @pl.when(lax.rem(k, NBUF) == bb)
                def _(bb=bb):
                    nxt = k + DG
                    bn = (bb + DG) % NBUF

                    @pl.when(nxt < valid)
                    def _():
                        @pl.when(nxt >= NBUF)
                        def _():
                            wait_write(bn)
                        start_read(nxt, bn)

                    wait_read(bb)
                    transpose(bb)
                    start_write(k, bb)
            return carry

        lax.fori_loop(0, valid, body, 0)

        for j in range(NBUF):
            @pl.when(valid > j)
            def _(j=j):
                for bb in range(NBUF):
                    @pl.when(lax.rem(valid - 1 - j, NBUF) == bb)
                    def _(bb=bb):
                        wait_write(bb)

        if TAIL:
            @pl.when(wid == 0)
            def _():
                pltpu.async_copy(
                    tokT_hbm.at[:, pl.ds(NCHUNK * CW, TAIL)], tin, gsems[0]
                ).wait()

                @plsc.parallel_loop(0, TAIL // LANES, unroll=2)
                def tail_body(m):
                    t_vec = m * LANES + iota
                    t32 = t_vec * 32
                    for rot in range(32):
                        e_vec = (iota + rot) & 31
                        vals = plsc.load_gather(tin, [e_vec, t_vec])
                        flat = t32 + e_vec
                        plsc.store_scatter(
                            tout, [flat >> 7, flat & 127], vals)

                pltpu.async_copy(
                    tout,
                    out_hbm.at[pl.ds(NCHUNK * CW // 4, TAIL * 32 // 128)],
                    gsems[0],
                ).wait()

    return detile


@functools.lru_cache(maxsize=None)
def _make_emb(batch, maxlen, embed, vocab):
    RPW = batch // NW
    TB = 4
    NBUF = 2
    DG = 1
    NSTEP = maxlen // TB
    assert RPW == 128 and embed == 32 and maxlen % TB == 0

    mesh = plsc.VectorSubcoreMesh(core_axis_name="c", subcore_axis_name="s")

    @functools.partial(
        pl.kernel,
        mesh=mesh,
        compiler_params=pltpu.CompilerParams(use_tc_tiling_on_sc=False,
                                             needs_layout_passes=False),
        out_type=jax.ShapeDtypeStruct((maxlen, embed // 8, batch // 128, 8, 128),
                                      jnp.float32),
        scratch_types=(
            [pltpu.VMEM((maxlen, RPW), jnp.int32),
             pltpu.VMEM((NSTEP, TB * RPW), jnp.int32),
             pltpu.VMEM((maxlen, embed), jnp.float32)]
            + [pltpu.VMEM((TB * RPW, embed), jnp.float32) for _ in range(NBUF)]
            + [pltpu.VMEM((TB, embed // 8, 1, 8, 128), jnp.float32)
               for _ in range(NBUF)]
            + [pltpu.SemaphoreType.DMA for _ in range(2 * NBUF + 1)]
        ),
    )
    def emb(xT_hbm, tok_hbm, pos_hbm, out_hbm, idx2d_v, idxT_v, pat_v, *rest):
        gbuf = rest[:NBUF]
        obuf = rest[NBUF:2 * NBUF]
        gsems = rest[2 * NBUF:3 * NBUF]
        ssems = rest[3 * NBUF:4 * NBUF]
        lsem = rest[4 * NBUF]

        wid = lax.axis_index("s") * NC + lax.axis_index("c")
        base = wid * RPW
        iota = _iota16()
        b_vecs = [j * LANES + iota for j in range(RPW // LANES)]

        pltpu.async_copy(xT_hbm.at[:, pl.ds(base, RPW)], idx2d_v, lsem).wait()
        pltpu.async_copy(pos_hbm, pat_v, lsem).wait()

        @plsc.parallel_loop(0, maxlen, unroll=8)
        def repack(t):
            o = (t & (TB - 1)) * RPW
            for j in range(RPW // LANES):
                idxT_v[t >> 2, pl.ds(o + j * LANES, LANES)] = (
                    idx2d_v[t, pl.ds(j * LANES, LANES)])

        def start_gather(s, b):
            pltpu.make_async_copy(
                tok_hbm.at[idxT_v.at[s]], gbuf[b], gsems[b]
            ).start()

        def wait_gather(b):
            pltpu.make_async_copy(
                tok_hbm.at[idxT_v.at[0]], gbuf[b], gsems[b]
            ).wait()

        def start_scatter(s, b):
            pltpu.make_async_copy(
                obuf[b], out_hbm.at[pl.ds(s * TB, TB), :, pl.ds(wid, 1)],
                ssems[b],
            ).start()

        def wait_scatter(b):
            pltpu.make_async_copy(
                obuf[b], out_hbm.at[pl.ds(0, TB), :, pl.ds(wid, 1)], ssems[b]
            ).wait()

        zeros = jnp.zeros((LANES,), jnp.int32)

        def transpose_add(s, b):
            @plsc.parallel_loop(0, TB * RPW // LANES, unroll=2)
            def body(m):
                tt = m >> 3
                row_vec = m * LANES + iota
                c_vec = (m & 7) * LANES + iota
                t_vec = jnp.full((LANES,), s * TB + tt, jnp.int32)
                tt_vec = jnp.full((LANES,), tt, jnp.int32)
                for rot in range(32):
                    e_vec = (iota + rot) & 31
                    vals = plsc.load_gather(gbuf[b], [row_vec, e_vec])
                    pv = plsc.load_gather(pat_v, [t_vec, e_vec])
                    plsc.store_scatter(
                        obuf[b],
                        [tt_vec, e_vec >> 3, zeros, e_vec & 7, c_vec],
                        vals + pv)

        for b in range(DG):
            start_gather(b, b)

        def outer(i, carry):
            s0 = i * NBUF
            for b in range(NBUF):
                s = s0 + b
                nxt = s + DG
                bn = (b + DG) % NBUF

                @pl.when(nxt < NSTEP)
                def _(nxt=nxt, bn=bn):
                    @pl.when(nxt >= NBUF)
                    def _():
                        wait_scatter(bn)
                    start_gather(nxt, bn)

                wait_gather(b)
                transpose_add(s, b)
                start_scatter(s, b)
            return carry

        lax.fori_loop(0, NSTEP // NBUF, outer, 0)

        for b in range(NBUF):
            wait_scatter(b)

    return emb


def kernel(x, token_table, pos_table):
    batch, maxlen = x.shape
    vocab, embed = token_table.shape
    tok_dense = _make_detile(vocab, embed)(token_table.T)
    tok_lin = tok_dense.reshape(vocab, embed)
    out5 = _make_emb(batch, maxlen, embed, vocab)(
        x.astype(jnp.int32).T, tok_lin, pos_table
    )
    return out5.transpose(2, 4, 0, 1, 3).reshape(batch, maxlen, embed)


# speedup vs baseline: 3.3648x; 1.7947x over previous
import functools

import jax
import jax.numpy as jnp
from jax import lax
from jax.experimental import pallas as pl
from jax.experimental.pallas import tpu as pltpu
from jax.experimental.pallas import tpu_sc as plsc

LANES = 16
NC = 2
NS = 16
NW = NC * NS


def _iota16():
    return lax.iota(jnp.int32, LANES)


@functools.lru_cache(maxsize=None)
def _make_detile(vocab, embed):
    KC = 4
    CW = 128 * KC
    TCOLS = vocab // 128
    NCHUNK = TCOLS // KC
    TAIL = vocab - NCHUNK * CW
    NBUF = 3
    DG = 1
    assert embed == 32 and TAIL % 4 == 0
    mesh = plsc.VectorSubcoreMesh(core_axis_name="c", subcore_axis_name="s")

    @functools.partial(
        pl.kernel,
        mesh=mesh,
        compiler_params=pltpu.CompilerParams(needs_layout_passes=False),
        out_type=jax.ShapeDtypeStruct((vocab * embed // 128, 128), jnp.float32),
        scratch_types=(
            [pltpu.VMEM((32, CW), jnp.float32) for _ in range(NBUF)]
            + [pltpu.VMEM((CW // 4, 128), jnp.float32) for _ in range(NBUF)]
            + ([pltpu.VMEM((32, TAIL), jnp.float32),
                pltpu.VMEM((TAIL * 32 // 128, 128), jnp.float32)] if TAIL else [])
            + [pltpu.SemaphoreType.DMA for _ in range(2 * NBUF)]
        ),
    )
    def detile(tokT_hbm, out_hbm, *rest):
        vin = rest[:NBUF]
        vout = rest[NBUF:2 * NBUF]
        ntail = 2 if TAIL else 0
        if TAIL:
            tin, tout = rest[2 * NBUF:2 * NBUF + 2]
        gsems = rest[2 * NBUF + ntail:3 * NBUF + ntail]
        ssems = rest[3 * NBUF + ntail:4 * NBUF + ntail]

        wid = lax.axis_index("s") * NC + lax.axis_index("c")
        iota = _iota16()
        e_vecs = ((0, iota), (1, iota + LANES))

        def chunk_of(k):
            return k * NW + wid

        def start_read(k, b):
            pltpu.make_async_copy(
                tokT_hbm.at[:, pl.ds(chunk_of(k) * CW, CW)], vin[b], gsems[b]
            ).start()

        def wait_read(b):
            pltpu.make_async_copy(
                tokT_hbm.at[:, pl.ds(0, CW)], vin[b], gsems[b]
            ).wait()

        def start_write(k, b):
            pltpu.make_async_copy(
                vout[b], out_hbm.at[pl.ds(chunk_of(k) * (CW // 4), CW // 4)],
                ssems[b],
            ).start()

        def wait_write(b):
            pltpu.make_async_copy(
                vout[b], out_hbm.at[pl.ds(0, CW // 4)], ssems[b]
            ).wait()

        def transpose(b):
            @plsc.parallel_loop(0, CW // LANES, unroll=2)
            def body(m):
                t_vec = m * LANES + iota
                t32 = t_vec * 32
                for rot in range(32):
                    e_vec = (iota + rot) & 31
                    vals = plsc.load_gather(vin[b], [e_vec, t_vec])
                    flat = t32 + e_vec
                    plsc.store_scatter(
                        vout[b], [flat >> 7, flat & 127], vals)

        valid = (NCHUNK - 1 - wid) // NW + 1

        for b in range(DG):
            @pl.when(b < valid)
            def _(b=b):
                start_read(b, b)

        def body(k, carry):
            for bb in range(NBUF):
                @pl.when(lax.rem(k, NBUF) == bb)
                def _(bb=bb):
                    nxt = k + DG
                    bn = (bb + DG) % NBUF

                    @pl.when(nxt < valid)
                    def _():
                        @pl.when(nxt >= NBUF)
                        def _():
                            wait_write(bn)
                        start_read(nxt, bn)

                    wait_read(bb)
                    transpose(bb)
                    start_write(k, bb)
            return carry

        lax.fori_loop(0, valid, body, 0)

        for j in range(NBUF):
            @pl.when(valid > j)
            def _(j=j):
                for bb in range(NBUF):
                    @pl.when(lax.rem(valid - 1 - j, NBUF) == bb)
                    def _(bb=bb):
                        wait_write(bb)

        if TAIL:
            @pl.when(wid == 0)
            def _():
                pltpu.async_copy(
                    tokT_hbm.at[:, pl.ds(NCHUNK * CW, TAIL)], tin, gsems[0]
                ).wait()

                @plsc.parallel_loop(0, TAIL // LANES, unroll=2)
                def tail_body(m):
                    t_vec = m * LANES + iota
                    t32 = t_vec * 32
                    for rot in range(32):
                        e_vec = (iota + rot) & 31
                        vals = plsc.load_gather(tin, [e_vec, t_vec])
                        flat = t32 + e_vec
                        plsc.store_scatter(
                            tout, [flat >> 7, flat & 127], vals)

                pltpu.async_copy(
                    tout,
                    out_hbm.at[pl.ds(NCHUNK * CW // 4, TAIL * 32 // 128)],
                    gsems[0],
                ).wait()

    return detile


@functools.lru_cache(maxsize=None)
def _make_emb(batch, maxlen, embed, vocab):
    RPW = batch // NW
    TB = 4
    NBUF = 2
    DG = 1
    NSTEP = maxlen // TB
    assert RPW == 128 and embed == 32 and maxlen % TB == 0

    mesh = plsc.VectorSubcoreMesh(core_axis_name="c", subcore_axis_name="s")

    @functools.partial(
        pl.kernel,
        mesh=mesh,
        compiler_params=pltpu.CompilerParams(use_tc_tiling_on_sc=False,
                                             needs_layout_passes=False),
        out_type=jax.ShapeDtypeStruct((maxlen, embed // 8, batch // 128, 8, 128),
                                      jnp.float32),
        scratch_types=(
            [pltpu.VMEM((maxlen, RPW), jnp.int32),
             pltpu.VMEM((NSTEP, TB * RPW), jnp.int32),
             pltpu.VMEM((maxlen, embed), jnp.float32)]
            + [pltpu.VMEM((TB * RPW, embed), jnp.float32) for _ in range(NBUF)]
            + [pltpu.VMEM((TB, embed // 8, 1, 8, 128), jnp.float32)
               for _ in range(NBUF)]
            + [pltpu.SemaphoreType.DMA for _ in range(2 * NBUF + 1)]
        ),
    )
    def emb(xT_hbm, tok_hbm, pos_hbm, out_hbm, idx2d_v, idxT_v, pat_v, *rest):
        gbuf = rest[:NBUF]
        obuf = rest[NBUF:2 * NBUF]
        gsems = rest[2 * NBUF:3 * NBUF]
        ssems = rest[3 * NBUF:4 * NBUF]
        lsem = rest[4 * NBUF]

        wid = lax.axis_index("s") * NC + lax.axis_index("c")
        base = wid * RPW
        iota = _iota16()
        b_vecs = [j * LANES + iota for j in range(RPW // LANES)]

        pltpu.async_copy(xT_hbm.at[:, pl.ds(base, RPW)], idx2d_v, lsem).wait()
        pltpu.async_copy(pos_hbm, pat_v, lsem).wait()

        @plsc.parallel_loop(0, maxlen, unroll=8)
        def repack(t):
            o = (t & (TB - 1)) * RPW
            for j in range(RPW // LANES):
                idxT_v[t >> 2, pl.ds(o + j * LANES, LANES)] = (
                    idx2d_v[t, pl.ds(j * LANES, LANES)])

        def start_gather(s, b):
            pltpu.make_async_copy(
                tok_hbm.at[idxT_v.at[s]], gbuf[b], gsems[b]
            ).start()

        def wait_gather(b):
            pltpu.make_async_copy(
                tok_hbm.at[idxT_v.at[0]], gbuf[b], gsems[b]
            ).wait()

        def start_scatter(s, b):
            pltpu.make_async_copy(
                obuf[b], out_hbm.at[pl.ds(s * TB, TB), :, pl.ds(wid, 1)],
                ssems[b],
            ).start()

        def wait_scatter(b):
            pltpu.make_async_copy(
                obuf[b], out_hbm.at[pl.ds(0, TB), :, pl.ds(wid, 1)], ssems[b]
            ).wait()

        zeros = jnp.zeros((LANES,), jnp.int32)

        def transpose_add(s, b):
            @plsc.parallel_loop(0, TB * 32, unroll=2)
            def body(j):
                tt = j >> 5
                rot = j & 31
                e_vec = (iota + rot) & 31
                t_vec = jnp.full((LANES,), s * TB + tt, jnp.int32)
                tt_vec = jnp.full((LANES,), tt, jnp.int32)
                pv = plsc.load_gather(pat_v, [t_vec, e_vec])
                te_vec = e_vec >> 3
                r_vec = e_vec & 7
                row0 = tt * RPW + iota
                for g in range(RPW // LANES):
                    row_vec = row0 + g * LANES
                    vals = plsc.load_gather(gbuf[b], [row_vec, e_vec])
                    plsc.store_scatter(
                        obuf[b],
                        [tt_vec, te_vec, zeros, r_vec, b_vecs[g]],
                        vals + pv)

        for b in range(DG):
            start_gather(b, b)

        def outer(i, carry):
            s0 = i * NBUF
            for b in range(NBUF):
                s = s0 + b
                nxt = s + DG
                bn = (b + DG) % NBUF

                @pl.when(nxt < NSTEP)
                def _(nxt=nxt, bn=bn):
                    @pl.when(nxt >= NBUF)
                    def _():
                        wait_scatter(bn)
                    start_gather(nxt, bn)

                wait_gather(b)
                transpose_add(s, b)
                start_scatter(s, b)
            return carry

        lax.fori_loop(0, NSTEP // NBUF, outer, 0)

        for b in range(NBUF):
            wait_scatter(b)

    return emb


def kernel(x, token_table, pos_table):
    batch, maxlen = x.shape
    vocab, embed = token_table.shape
    tok_dense = _make_detile(vocab, embed)(token_table.T)
    tok_lin = tok_dense.reshape(vocab, embed)
    out5 = _make_emb(batch, maxlen, embed, vocab)(
        x.astype(jnp.int32).T, tok_lin, pos_table
    )
    return out5.transpose(2, 4, 0, 1, 3).reshape(batch, maxlen, embed)


# speedup vs baseline: 3.5194x; 1.0459x over previous
import functools

import jax
import jax.numpy as jnp
from jax import lax
from jax.experimental import pallas as pl
from jax.experimental.pallas import tpu as pltpu
from jax.experimental.pallas import tpu_sc as plsc

LANES = 16
NC = 2
NS = 16
NW = NC * NS


def _iota16():
    return lax.iota(jnp.int32, LANES)


@functools.lru_cache(maxsize=None)
def _make_detile(vocab, embed):
    KC = 4
    CW = 128 * KC
    TCOLS = vocab // 128
    NCHUNK = TCOLS // KC
    TAIL = vocab - NCHUNK * CW
    NBUF = 3
    DG = 1
    assert embed == 32 and TAIL % 4 == 0
    mesh = plsc.VectorSubcoreMesh(core_axis_name="c", subcore_axis_name="s")

    @functools.partial(
        pl.kernel,
        mesh=mesh,
        compiler_params=pltpu.CompilerParams(needs_layout_passes=False),
        out_type=jax.ShapeDtypeStruct((vocab * embed // 128, 128), jnp.float32),
        scratch_types=(
            [pltpu.VMEM((32, CW), jnp.float32) for _ in range(NBUF)]
            + [pltpu.VMEM((CW // 4, 128), jnp.float32) for _ in range(NBUF)]
            + ([pltpu.VMEM((32, TAIL), jnp.float32),
                pltpu.VMEM((TAIL * 32 // 128, 128), jnp.float32)] if TAIL else [])
            + [pltpu.SemaphoreType.DMA for _ in range(2 * NBUF)]
        ),
    )
    def detile(tokT_hbm, out_hbm, *rest):
        vin = rest[:NBUF]
        vout = rest[NBUF:2 * NBUF]
        ntail = 2 if TAIL else 0
        if TAIL:
            tin, tout = rest[2 * NBUF:2 * NBUF + 2]
        gsems = rest[2 * NBUF + ntail:3 * NBUF + ntail]
        ssems = rest[3 * NBUF + ntail:4 * NBUF + ntail]

        wid = lax.axis_index("s") * NC + lax.axis_index("c")
        iota = _iota16()
        e_vecs = ((0, iota), (1, iota + LANES))

        def chunk_of(k):
            return k * NW + wid

        def start_read(k, b):
            pltpu.make_async_copy(
                tokT_hbm.at[:, pl.ds(chunk_of(k) * CW, CW)], vin[b], gsems[b]
            ).start()

        def wait_read(b):
            pltpu.make_async_copy(
                tokT_hbm.at[:, pl.ds(0, CW)], vin[b], gsems[b]
            ).wait()

        def start_write(k, b):
            pltpu.make_async_copy(
                vout[b], out_hbm.at[pl.ds(chunk_of(k) * (CW // 4), CW // 4)],
                ssems[b],
            ).start()

        def wait_write(b):
            pltpu.make_async_copy(
                vout[b], out_hbm.at[pl.ds(0, CW // 4)], ssems[b]
            ).wait()

        def transpose(b):
            @plsc.parallel_loop(0, 32, unroll=2)
            def body(rot):
                e_vec = (iota + rot) & 31
                for m in range(CW // LANES):
                    t_vec = m * LANES + iota
                    vals = plsc.load_gather(vin[b], [e_vec, t_vec])
                    flat = t_vec * 32 + e_vec
                    plsc.store_scatter(
                        vout[b], [flat >> 7, flat & 127], vals)

        valid = (NCHUNK - 1 - wid) // NW + 1

        for b in range(DG):
            @pl.when(b < valid)
            def _(b=b):
                start_read(b, b)

        def body(k, carry):
            for bb in range(NBUF):
                @pl.when(lax.rem(k, NBUF) == bb)
                def _(bb=bb):
                    nxt = k + DG
                    bn = (bb + DG) % NBUF

                    @pl.when(nxt < valid)
                    def _():
                        @pl.when(nxt >= NBUF)
                        def _():
                            wait_write(bn)
                        start_read(nxt, bn)

                    wait_read(bb)
                    transpose(bb)
                    start_write(k, bb)
            return carry

        lax.fori_loop(0, valid, body, 0)

        for j in range(NBUF):
            @pl.when(valid > j)
            def _(j=j):
                for bb in range(NBUF):
                    @pl.when(lax.rem(valid - 1 - j, NBUF) == bb)
                    def _(bb=bb):
                        wait_write(bb)

        if TAIL:
            @pl.when(wid == 0)
            def _():
                pltpu.async_copy(
                    tokT_hbm.at[:, pl.ds(NCHUNK * CW, TAIL)], tin, gsems[0]
                ).wait()

                @plsc.parallel_loop(0, TAIL // LANES, unroll=2)
                def tail_body(m):
                    t_vec = m * LANES + iota
                    t32 = t_vec * 32
                    for rot in range(32):
                        e_vec = (iota + rot) & 31
                        vals = plsc.load_gather(tin, [e_vec, t_vec])
                        flat = t32 + e_vec
                        plsc.store_scatter(
                            tout, [flat >> 7, flat & 127], vals)

                pltpu.async_copy(
                    tout,
                    out_hbm.at[pl.ds(NCHUNK * CW // 4, TAIL * 32 // 128)],
                    gsems[0],
                ).wait()

    return detile


@functools.lru_cache(maxsize=None)
def _make_emb(batch, maxlen, embed, vocab):
    RPW = batch // NW
    TB = 4
    NBUF = 2
    DG = 1
    NSTEP = maxlen // TB
    assert RPW == 128 and embed == 32 and maxlen % TB == 0

    mesh = plsc.VectorSubcoreMesh(core_axis_name="c", subcore_axis_name="s")

    @functools.partial(
        pl.kernel,
        mesh=mesh,
        compiler_params=pltpu.CompilerParams(use_tc_tiling_on_sc=False,
                                             needs_layout_passes=False),
        out_type=jax.ShapeDtypeStruct((maxlen, embed // 8, batch // 128, 8, 128),
                                      jnp.float32),
        scratch_types=(
            [pltpu.VMEM((maxlen, RPW), jnp.int32),
             pltpu.VMEM((NSTEP, TB * RPW), jnp.int32),
             pltpu.VMEM((maxlen, embed), jnp.float32)]
            + [pltpu.VMEM((TB * RPW, embed), jnp.float32) for _ in range(NBUF)]
            + [pltpu.VMEM((TB, embed // 8, 1, 8, 128), jnp.float32)
               for _ in range(NBUF)]
            + [pltpu.SemaphoreType.DMA for _ in range(2 * NBUF + 1)]
        ),
    )
    def emb(xT_hbm, tok_hbm, pos_hbm, out_hbm, idx2d_v, idxT_v, pat_v, *rest):
        gbuf = rest[:NBUF]
        obuf = rest[NBUF:2 * NBUF]
        gsems = rest[2 * NBUF:3 * NBUF]
        ssems = rest[3 * NBUF:4 * NBUF]
        lsem = rest[4 * NBUF]

        wid = lax.axis_index("s") * NC + lax.axis_index("c")
        base = wid * RPW
        iota = _iota16()
        b_vecs = [j * LANES + iota for j in range(RPW // LANES)]

        pltpu.async_copy(xT_hbm.at[:, pl.ds(base, RPW)], idx2d_v, lsem).wait()
        pltpu.async_copy(pos_hbm, pat_v, lsem).wait()

        @plsc.parallel_loop(0, maxlen, unroll=8)
        def repack(t):
            o = (t & (TB - 1)) * RPW
            for j in range(RPW // LANES):
                idxT_v[t >> 2, pl.ds(o + j * LANES, LANES)] = (
                    idx2d_v[t, pl.ds(j * LANES, LANES)])

        def start_gather(s, b):
            pltpu.make_async_copy(
                tok_hbm.at[idxT_v.at[s]], gbuf[b], gsems[b]
            ).start()

        def wait_gather(b):
            pltpu.make_async_copy(
                tok_hbm.at[idxT_v.at[0]], gbuf[b], gsems[b]
            ).wait()

        def start_scatter(s, b):
            pltpu.make_async_copy(
                obuf[b], out_hbm.at[pl.ds(s * TB, TB), :, pl.ds(wid, 1)],
                ssems[b],
            ).start()

        def wait_scatter(b):
            pltpu.make_async_copy(
                obuf[b], out_hbm.at[pl.ds(0, TB), :, pl.ds(wid, 1)], ssems[b]
            ).wait()

        zeros = jnp.zeros((LANES,), jnp.int32)

        def transpose_add(s, b):
            @plsc.parallel_loop(0, TB * 32, unroll=2)
            def body(j):
                tt = j >> 5
                rot = j & 31
                e_vec = (iota + rot) & 31
                t_vec = jnp.full((LANES,), s * TB + tt, jnp.int32)
                tt_vec = jnp.full((LANES,), tt, jnp.int32)
                pv = plsc.load_gather(pat_v, [t_vec, e_vec])
                te_vec = e_vec >> 3
                r_vec = e_vec & 7
                row0 = tt * RPW + iota
                for g in range(RPW // LANES):
                    row_vec = row0 + g * LANES
                    vals = plsc.load_gather(gbuf[b], [row_vec, e_vec])
                    plsc.store_scatter(
                        obuf[b],
                        [tt_vec, te_vec, zeros, r_vec, b_vecs[g]],
                        vals + pv)

        for b in range(DG):
            start_gather(b, b)

        def outer(i, carry):
            s0 = i * NBUF
            for b in range(NBUF):
                s = s0 + b
                nxt = s + DG
                bn = (b + DG) % NBUF

                @pl.when(nxt < NSTEP)
                def _(nxt=nxt, bn=bn):
                    @pl.when(nxt >= NBUF)
                    def _():
                        wait_scatter(bn)
                    start_gather(nxt, bn)

                wait_gather(b)
                transpose_add(s, b)
                start_scatter(s, b)
            return carry

        lax.fori_loop(0, NSTEP // NBUF, outer, 0)

        for b in range(NBUF):
            wait_scatter(b)

    return emb


def kernel(x, token_table, pos_table):
    batch, maxlen = x.shape
    vocab, embed = token_table.shape
    tok_dense = _make_detile(vocab, embed)(token_table.T)
    tok_lin = tok_dense.reshape(vocab, embed)
    out5 = _make_emb(batch, maxlen, embed, vocab)(
        x.astype(jnp.int32).T, tok_lin, pos_table
    )
    return out5.transpose(2, 4, 0, 1, 3).reshape(batch, maxlen, embed)
